# Initial kernel scaffold; baseline (speedup 1.0000x reference)
import jax
import jax.numpy as jnp
from jax.experimental import pallas as pl


def kernel(xyz, params):
    raise NotImplementedError("write your pallas kernel here")


# speedup vs baseline: 1.0000x; 1.0000x over previous
import functools

import jax
import jax.numpy as jnp
import numpy as np
from jax.experimental import pallas as pl
from jax.experimental.pallas import tpu as pltpu

BN_SCALE = 1.0 / np.sqrt(1.0 + 1e-5)


def _square_distance(src, dst):
    return (jnp.sum(src ** 2, -1)[:, :, None]
            + jnp.sum(dst ** 2, -1)[:, None, :]
            - 2.0 * jnp.matmul(src, dst.transpose(0, 2, 1)))


def _index_points(points, idx):
    B = points.shape[0]
    batch_idx = jnp.arange(B).reshape((B,) + (1,) * (idx.ndim - 1))
    return points[batch_idx, idx]


def _farthest_point_sample(xyz, npoint):
    xyz = jax.lax.stop_gradient(xyz)
    B, N, _ = xyz.shape
    def step(state, _):
        distance, farthest = state
        centroid = xyz[jnp.arange(B), farthest][:, None, :]
        d = jnp.sum((xyz - centroid) ** 2, -1)
        distance = jnp.minimum(distance, d)
        nxt = jnp.argmax(distance, -1).astype(jnp.int32)
        return (distance, nxt), farthest
    init = (jnp.full((B, N), 1e10, jnp.float32), jnp.zeros((B,), jnp.int32))
    _, cents = jax.lax.scan(step, init, None, length=npoint)
    return cents.T


def _query_ball_point(radius, nsample, xyz, new_xyz):
    B, N, _ = xyz.shape
    S = new_xyz.shape[1]
    sqrdists = jax.lax.stop_gradient(_square_distance(new_xyz, xyz))
    gi = jnp.broadcast_to(jnp.arange(N, dtype=jnp.int32), (B, S, N))
    gi = jnp.where(sqrdists > radius ** 2, N, gi)
    gi = jnp.sort(gi, axis=-1)[:, :, :nsample]
    first = jnp.broadcast_to(gi[:, :, :1], gi.shape)
    return jnp.where(gi == N, first, gi)


def _conv_bn_relu_2d(gp, layers):
    for W, b in layers:
        gp = jnp.einsum('oi,biks->boks', W, gp) + b[None, :, None, None]
        gp = jax.nn.relu(gp * BN_SCALE)
    return gp


def _sa_msg(xyz, points, npoint, radii, nsamples, branches):
    xyz_t = xyz.transpose(0, 2, 1)
    points_t = points.transpose(0, 2, 1)
    fps_idx = _farthest_point_sample(xyz_t, npoint)
    new_xyz = _index_points(xyz_t, fps_idx)
    outs = []
    for radius, K, layers in zip(radii, nsamples, branches):
        gidx = _query_ball_point(radius, K, xyz_t, new_xyz)
        g_xyz = _index_points(xyz_t, gidx) - new_xyz[:, :, None, :]
        g_pts = jnp.concatenate([_index_points(points_t, gidx), g_xyz], axis=-1)
        gp = g_pts.transpose(0, 3, 2, 1)
        gp = _conv_bn_relu_2d(gp, layers)
        outs.append(jnp.max(gp, axis=2))
    return new_xyz.transpose(0, 2, 1), jnp.concatenate(outs, axis=1)


def _sa_group_all(xyz, points, layers):
    xyz_t = xyz.transpose(0, 2, 1)
    points_t = points.transpose(0, 2, 1)
    B = xyz_t.shape[0]
    grouped = jnp.concatenate([xyz_t, points_t], axis=-1)[:, None, :, :]
    gp = grouped.transpose(0, 3, 2, 1)
    gp = _conv_bn_relu_2d(gp, layers)
    new_points = jnp.max(gp, axis=2)
    new_xyz = jnp.zeros((B, 3, 1), jnp.float32)
    return new_xyz, new_points


def _feature_propagation(xyz1, xyz2, points1, points2, layers):
    xyz1_t = xyz1.transpose(0, 2, 1)
    xyz2_t = xyz2.transpose(0, 2, 1)
    points2_t = points2.transpose(0, 2, 1)
    B, N, _ = xyz1_t.shape
    S = xyz2_t.shape[1]
    if S == 1:
        interp = jnp.broadcast_to(points2_t, (B, N, points2_t.shape[-1]))
    else:
        dists = _square_distance(xyz1_t, xyz2_t)
        idx = jnp.argsort(dists, axis=-1)[:, :, :3]
        d3 = jnp.take_along_axis(dists, idx, axis=-1)
        w = 1.0 / (d3 + 1e-8)
        w = w / jnp.sum(w, axis=2, keepdims=True)
        interp = jnp.sum(_index_points(points2_t, idx) * w[..., None], axis=2)
    points1_t = points1.transpose(0, 2, 1)
    new_points = jnp.concatenate([points1_t, interp], axis=-1)
    gp = new_points.transpose(0, 2, 1)
    for W, b in layers:
        gp = jnp.einsum('oi,bin->bon', W, gp) + b[None, :, None]
        gp = jax.nn.relu(gp * BN_SCALE)
    return gp


def _head_kernel(x_ref, w1_ref, b1_ref, w2_ref, b2_ref, o_ref):
    x = x_ref[0]
    h = jnp.dot(w1_ref[...], x, preferred_element_type=jnp.float32)
    h = jax.nn.relu((h + b1_ref[...][:, None]) * BN_SCALE)
    y = jnp.dot(w2_ref[...], h, preferred_element_type=jnp.float32)
    y = y + b2_ref[...][:, None]
    o_ref[0] = jax.nn.log_softmax(y, axis=0)


def _head(l0_points, params):
    B, C, N = l0_points.shape
    W1, b1 = params['conv1']
    W2, b2 = params['conv2']
    out = pl.pallas_call(
        _head_kernel,
        out_shape=jax.ShapeDtypeStruct((B, 13, N), jnp.float32),
        grid=(B,),
        in_specs=[
            pl.BlockSpec((1, C, N), lambda b: (b, 0, 0)),
            pl.BlockSpec((W1.shape[0], W1.shape[1]), lambda b: (0, 0)),
            pl.BlockSpec((W1.shape[0],), lambda b: (0,)),
            pl.BlockSpec((W2.shape[0], W2.shape[1]), lambda b: (0, 0)),
            pl.BlockSpec((W2.shape[0],), lambda b: (0,)),
        ],
        out_specs=pl.BlockSpec((1, 13, N), lambda b: (b, 0, 0)),
    )(l0_points, W1, b1, W2, b2)
    return out


def kernel(xyz, params):
    l0_points = xyz
    l0_xyz = xyz
    l1_xyz, l1_points = _sa_msg(l0_xyz, l0_points, 512, [0.1, 0.2, 0.4], [32, 64, 128], params['sa1'])
    l2_xyz, l2_points = _sa_msg(l1_xyz, l1_points, 128, [0.4, 0.8], [64, 128], params['sa2'])
    l3_xyz, l3_points = _sa_group_all(l2_xyz, l2_points, params['sa3'])
    l2_points = _feature_propagation(l2_xyz, l3_xyz, l2_points, l3_points, params['fp3'])
    l1_points = _feature_propagation(l1_xyz, l2_xyz, l1_points, l2_points, params['fp2'])
    l0_points = _feature_propagation(xyz, l1_xyz, l0_points, l1_points, params['fp1'])
    x = _head(l0_points, params)
    return x.transpose(0, 2, 1), l3_points


# speedup vs baseline: 1.0484x; 1.0484x over previous
import functools

import jax
import jax.numpy as jnp
import numpy as np
from jax.experimental import pallas as pl
from jax.experimental.pallas import tpu as pltpu

BN_SCALE = 1.0 / np.sqrt(1.0 + 1e-5)


def _fps_body(npoint, x_ref, y_ref, z_ref, ox_ref, oy_ref, oz_ref):
    B, N = x_ref.shape
    x = x_ref[...]; y = y_ref[...]; z = z_ref[...]
    iota_n = jax.lax.broadcasted_iota(jnp.int32, (B, N), 1)
    iota_p = jax.lax.broadcasted_iota(jnp.int32, (B, npoint), 1)

    def step(i, carry):
        dist, far, ox, oy, oz = carry
        onehot = iota_n == far
        cx = jnp.sum(jnp.where(onehot, x, 0.0), axis=1, keepdims=True)
        cy = jnp.sum(jnp.where(onehot, y, 0.0), axis=1, keepdims=True)
        cz = jnp.sum(jnp.where(onehot, z, 0.0), axis=1, keepdims=True)
        dx = x - cx; dy = y - cy; dz = z - cz
        d = (dx * dx + dy * dy) + dz * dz
        dist = jnp.minimum(dist, d)
        m = jnp.max(dist, axis=1, keepdims=True)
        nxt = jnp.min(jnp.where(dist == m, iota_n, N), axis=1, keepdims=True)
        sel = iota_p == i
        ox = jnp.where(sel, cx, ox)
        oy = jnp.where(sel, cy, oy)
        oz = jnp.where(sel, cz, oz)
        return (dist, nxt, ox, oy, oz)

    init = (jnp.full((B, N), 1e10, jnp.float32),
            jnp.zeros((B, 1), jnp.int32),
            jnp.zeros((B, npoint), jnp.float32),
            jnp.zeros((B, npoint), jnp.float32),
            jnp.zeros((B, npoint), jnp.float32))
    _, _, ox, oy, oz = jax.lax.fori_loop(0, npoint, step, init)
    ox_ref[...] = ox
    oy_ref[...] = oy
    oz_ref[...] = oz


def _fps(xyz, npoint):
    B, _, N = xyz.shape
    out = pl.pallas_call(
        functools.partial(_fps_body, npoint),
        out_shape=[jax.ShapeDtypeStruct((B, npoint), jnp.float32)] * 3,
    )(xyz[:, 0], xyz[:, 1], xyz[:, 2])
    return jnp.stack(out, axis=-1)


def _sa_mlp_body(K, C, layers_n, *refs):
    g_ref = refs[0]
    c_ref = refs[1]
    wrefs = refs[2:2 + 2 * layers_n]
    o_ref = refs[2 + 2 * layers_n]
    Sb = g_ref.shape[1]
    g3 = g_ref[0]
    ctr = c_ref[0]
    last3 = g3[:, :, C - 3:] - ctr[:, None, :]
    g3 = jnp.concatenate([g3[:, :, :C - 3], last3], axis=-1)
    h = g3.reshape(Sb * K, C)
    for li in range(layers_n):
        W = wrefs[2 * li][...]
        b = wrefs[2 * li + 1][...]
        h = jax.lax.dot_general(h, W, (((1,), (1,)), ((), ())),
                                preferred_element_type=jnp.float32)
        h = jax.nn.relu((h + b[None, :]) * BN_SCALE)
    Co = h.shape[-1]
    h = h.reshape(Sb, K, Co)
    o_ref[0] = jnp.max(h, axis=1)


def _sa_mlp(g3, ctr, layers, s_block):
    B, S, K, C = g3.shape
    Co = layers[-1][0].shape[0]
    wargs = []
    in_specs = [
        pl.BlockSpec((1, s_block, K, C), lambda b, s: (b, s, 0, 0)),
        pl.BlockSpec((1, s_block, 3), lambda b, s: (b, s, 0)),
    ]
    for W, bb in layers:
        wargs += [W, bb]
        in_specs += [pl.BlockSpec(W.shape, lambda b, s: (0, 0)),
                     pl.BlockSpec(bb.shape, lambda b, s: (0,))]
    out = pl.pallas_call(
        functools.partial(_sa_mlp_body, K, C, len(layers)),
        out_shape=jax.ShapeDtypeStruct((B, S, Co), jnp.float32),
        grid=(B, S // s_block),
        in_specs=in_specs,
        out_specs=pl.BlockSpec((1, s_block, Co), lambda b, s: (b, s, 0)),
    )(g3, ctr, *wargs)
    return out


def _sa3_fp3_body(*refs):
    rows_ref = refs[0]
    w = refs[1:11]
    l3_ref = refs[11]
    o_ref = refs[12]
    P = rows_ref.shape[1]
    h = rows_ref[0]
    for li in range(3):
        W = w[2 * li][...]
        b = w[2 * li + 1][...]
        h = jax.lax.dot_general(h, W, (((1,), (1,)), ((), ())),
                                preferred_element_type=jnp.float32)
        h = jax.nn.relu((h + b[None, :]) * BN_SCALE)
    l3 = jnp.max(h, axis=0)
    l3_ref[0, 0] = l3
    pts = rows_ref[0][:, 3:]
    h2 = jnp.concatenate([pts, jnp.broadcast_to(l3[None, :], (P, 1024))], axis=1)
    for li in range(2):
        W = w[6 + 2 * li][...]
        b = w[6 + 2 * li + 1][...]
        h2 = jax.lax.dot_general(h2, W, (((1,), (1,)), ((), ())),
                                 preferred_element_type=jnp.float32)
        h2 = jax.nn.relu((h2 + b[None, :]) * BN_SCALE)
    o_ref[0] = h2


def _sa3_fp3(l2_rows, sa3_layers, fp3_layers):
    B, P, C = l2_rows.shape
    wargs = []
    in_specs = [pl.BlockSpec((1, P, C), lambda b: (b, 0, 0))]
    for W, bb in sa3_layers + fp3_layers:
        wargs += [W, bb]
        in_specs += [pl.BlockSpec(W.shape, lambda b: (0, 0)),
                     pl.BlockSpec(bb.shape, lambda b: (0,))]
    l3, l2new = pl.pallas_call(
        _sa3_fp3_body,
        out_shape=[jax.ShapeDtypeStruct((B, 1, 1024), jnp.float32),
                   jax.ShapeDtypeStruct((B, P, 256), jnp.float32)],
        grid=(B,),
        in_specs=in_specs,
        out_specs=[pl.BlockSpec((1, 1, 1024), lambda b: (b, 0, 0)),
                   pl.BlockSpec((1, P, 256), lambda b: (b, 0, 0))],
    )(l2_rows, *wargs)
    return l3, l2new


def _fp_body(layers_n, *refs):
    xyz1_ref, xyz2t_ref, p1_ref, p2_ref = refs[:4]
    w = refs[4:4 + 2 * layers_n]
    o_ref = refs[4 + 2 * layers_n]
    x1 = xyz1_ref[0]
    x2t = xyz2t_ref[0]
    N = x1.shape[0]; S = x2t.shape[1]
    ns1 = jnp.sum(x1 * x1, axis=1, keepdims=True)
    ns2 = jnp.sum(x2t * x2t, axis=0, keepdims=True)
    mm = jax.lax.dot_general(x1, x2t, (((1,), (0,)), ((), ())),
                             preferred_element_type=jnp.float32)
    dists = (ns1 + ns2) - 2.0 * mm
    iota_s = jax.lax.broadcasted_iota(jnp.int32, (N, S), 1)
    cur = dists
    wmat = jnp.zeros((N, S), jnp.float32)
    wsum = jnp.zeros((N, 1), jnp.float32)
    onehots = []
    ws = []
    for _ in range(3):
        mk = jnp.min(cur, axis=1, keepdims=True)
        ik = jnp.min(jnp.where(cur == mk, iota_s, S), axis=1, keepdims=True)
        oh = iota_s == ik
        wk = 1.0 / (mk + 1e-8)
        onehots.append(oh)
        ws.append(wk)
        wsum = wsum + wk
        cur = jnp.where(oh, jnp.float32(np.inf), cur)
    for oh, wk in zip(onehots, ws):
        wmat = wmat + jnp.where(oh, (wk / wsum), 0.0)
    interp = jax.lax.dot_general(wmat, p2_ref[0], (((1,), (0,)), ((), ())),
                                 preferred_element_type=jnp.float32)
    h = jnp.concatenate([p1_ref[0], interp], axis=1)
    for li in range(layers_n):
        W = w[2 * li][...]
        b = w[2 * li + 1][...]
        h = jax.lax.dot_general(h, W, (((1,), (1,)), ((), ())),
                                preferred_element_type=jnp.float32)
        h = jax.nn.relu((h + b[None, :]) * BN_SCALE)
    o_ref[0] = h


def _fp(xyz1, xyz2, p1rows, p2rows, layers):
    B, N, _ = xyz1.shape
    S = xyz2.shape[1]
    C1 = p1rows.shape[2]; C2 = p2rows.shape[2]
    Co = layers[-1][0].shape[0]
    xyz2t = xyz2.transpose(0, 2, 1)
    wargs = []
    in_specs = [
        pl.BlockSpec((1, N, 3), lambda b: (b, 0, 0)),
        pl.BlockSpec((1, 3, S), lambda b: (b, 0, 0)),
        pl.BlockSpec((1, N, C1), lambda b: (b, 0, 0)),
        pl.BlockSpec((1, S, C2), lambda b: (b, 0, 0)),
    ]
    for W, bb in layers:
        wargs += [W, bb]
        in_specs += [pl.BlockSpec(W.shape, lambda b: (0, 0)),
                     pl.BlockSpec(bb.shape, lambda b: (0,))]
    out = pl.pallas_call(
        functools.partial(_fp_body, len(layers)),
        out_shape=jax.ShapeDtypeStruct((B, N, Co), jnp.float32),
        grid=(B,),
        in_specs=in_specs,
        out_specs=pl.BlockSpec((1, N, Co), lambda b: (b, 0, 0)),
    )(xyz1, xyz2t, p1rows, p2rows, *wargs)
    return out


def _head_body(x_ref, w1_ref, b1_ref, w2_ref, b2_ref, o_ref):
    x = x_ref[0]
    h = jax.lax.dot_general(x, w1_ref[...], (((1,), (1,)), ((), ())),
                            preferred_element_type=jnp.float32)
    h = jax.nn.relu((h + b1_ref[...][None, :]) * BN_SCALE)
    y = jax.lax.dot_general(h, w2_ref[...], (((1,), (1,)), ((), ())),
                            preferred_element_type=jnp.float32)
    y = y + b2_ref[...][None, :]
    m = jnp.max(y, axis=1, keepdims=True)
    sh = y - m
    lse = jnp.log(jnp.sum(jnp.exp(sh), axis=1, keepdims=True))
    o_ref[0] = sh - lse


def _head(rows, params):
    B, N, C = rows.shape
    W1, b1 = params['conv1']
    W2, b2 = params['conv2']
    out = pl.pallas_call(
        _head_body,
        out_shape=jax.ShapeDtypeStruct((B, N, 13), jnp.float32),
        grid=(B,),
        in_specs=[
            pl.BlockSpec((1, N, C), lambda b: (b, 0, 0)),
            pl.BlockSpec(W1.shape, lambda b: (0, 0)),
            pl.BlockSpec(b1.shape, lambda b: (0,)),
            pl.BlockSpec(W2.shape, lambda b: (0, 0)),
            pl.BlockSpec(b2.shape, lambda b: (0,)),
        ],
        out_specs=pl.BlockSpec((1, N, 13), lambda b: (b, 0, 0)),
    )(rows, W1, b1, W2, b2)
    return out


def _square_distance(src, dst):
    return (jnp.sum(src ** 2, -1)[:, :, None]
            + jnp.sum(dst ** 2, -1)[:, None, :]
            - 2.0 * jnp.matmul(src, dst.transpose(0, 2, 1)))


def _index_points(points, idx):
    B = points.shape[0]
    batch_idx = jnp.arange(B).reshape((B,) + (1,) * (idx.ndim - 1))
    return points[batch_idx, idx]


def _query_ball_point(radius, nsample, xyz, new_xyz):
    B, N, _ = xyz.shape
    S = new_xyz.shape[1]
    sqrdists = _square_distance(new_xyz, xyz)
    gi = jnp.broadcast_to(jnp.arange(N, dtype=jnp.int32), (B, S, N))
    gi = jnp.where(sqrdists > radius ** 2, N, gi)
    gi = jnp.sort(gi, axis=-1)[:, :, :nsample]
    first = jnp.broadcast_to(gi[:, :, :1], gi.shape)
    return jnp.where(gi == N, first, gi)


def _sa_msg(xyz_t, feat_rows, npoint, radii, nsamples, branches, s_blocks):
    B, N, _ = xyz_t.shape
    new_xyz = _fps(xyz_t.transpose(0, 2, 1), npoint)
    outs = []
    for radius, K, layers, sb in zip(radii, nsamples, branches, s_blocks):
        gidx = _query_ball_point(radius, K, xyz_t, new_xyz)
        g3 = _index_points(feat_rows, gidx)
        out = _sa_mlp(g3, new_xyz, layers, sb)
        outs.append(out)
    return new_xyz, jnp.concatenate(outs, axis=-1)


def kernel(xyz, params):
    B, _, N = xyz.shape
    xyz_t = xyz.transpose(0, 2, 1)
    feat0 = jnp.concatenate([xyz_t, xyz_t], axis=-1)
    l1_xyz, l1_rows = _sa_msg(xyz_t, feat0, 512, [0.1, 0.2, 0.4],
                              [32, 64, 128], params['sa1'],
                              [128, 128, 64])
    feat1 = jnp.concatenate([l1_rows, l1_xyz], axis=-1)
    l2_xyz, l2_rows = _sa_msg(l1_xyz, feat1, 128, [0.4, 0.8],
                              [64, 128], params['sa2'],
                              [64, 32])
    l2_all = jnp.concatenate([l2_xyz, l2_rows], axis=-1)
    l3_3d, l2new = _sa3_fp3(l2_all, params['sa3'], params['fp3'])
    l3 = l3_3d[:, 0]
    l1new = _fp(l1_xyz, l2_xyz, l1_rows, l2new, params['fp2'])
    l0new = _fp(xyz_t, l1_xyz, xyz_t, l1new, params['fp1'])
    logits = _head(l0new, params)
    return logits, l3[:, :, None]


# speedup vs baseline: 16.3748x; 15.6183x over previous
import functools

import jax
import jax.numpy as jnp
import numpy as np
from jax import lax
from jax.experimental import pallas as pl
from jax.experimental.pallas import tpu as pltpu
from jax.experimental.pallas import tpu_sc as plsc

BN_SCALE = 1.0 / np.sqrt(1.0 + 1e-5)


def _fps_body(npoint, x_ref, y_ref, z_ref, ox_ref, oy_ref, oz_ref):
    B, N = x_ref.shape
    x = x_ref[...]; y = y_ref[...]; z = z_ref[...]
    iota_n = jax.lax.broadcasted_iota(jnp.int32, (B, N), 1)
    iota_p = jax.lax.broadcasted_iota(jnp.int32, (B, npoint), 1)

    def step(i, carry):
        dist, far, ox, oy, oz = carry
        onehot = iota_n == far
        cx = jnp.sum(jnp.where(onehot, x, 0.0), axis=1, keepdims=True)
        cy = jnp.sum(jnp.where(onehot, y, 0.0), axis=1, keepdims=True)
        cz = jnp.sum(jnp.where(onehot, z, 0.0), axis=1, keepdims=True)
        dx = x - cx; dy = y - cy; dz = z - cz
        d = (dx * dx + dy * dy) + dz * dz
        dist = jnp.minimum(dist, d)
        m = jnp.max(dist, axis=1, keepdims=True)
        nxt = jnp.min(jnp.where(dist == m, iota_n, N), axis=1, keepdims=True)
        sel = iota_p == i
        ox = jnp.where(sel, cx, ox)
        oy = jnp.where(sel, cy, oy)
        oz = jnp.where(sel, cz, oz)
        return (dist, nxt, ox, oy, oz)

    init = (jnp.full((B, N), 1e10, jnp.float32),
            jnp.zeros((B, 1), jnp.int32),
            jnp.zeros((B, npoint), jnp.float32),
            jnp.zeros((B, npoint), jnp.float32),
            jnp.zeros((B, npoint), jnp.float32))
    _, _, ox, oy, oz = jax.lax.fori_loop(0, npoint, step, init)
    ox_ref[...] = ox
    oy_ref[...] = oy
    oz_ref[...] = oz


def _fps(xyz, npoint):
    B, _, N = xyz.shape
    out = pl.pallas_call(
        functools.partial(_fps_body, npoint),
        out_shape=[jax.ShapeDtypeStruct((B, npoint), jnp.float32)] * 3,
    )(xyz[:, 0], xyz[:, 1], xyz[:, 2])
    return jnp.stack(out, axis=-1)


def _sa_mlp_body(K, Cp, layers_n, *refs):
    g_ref = refs[0]
    c_ref = refs[1]
    wrefs = refs[2:2 + 2 * layers_n]
    o_ref = refs[2 + 2 * layers_n]
    Sb = g_ref.shape[1]
    g3 = g_ref[0]
    ctr = c_ref[0]
    xyz_c = g3[:, :, :3] - ctr[:, None, :]
    pts = g3[:, :, 3:3 + Cp]
    g3 = jnp.concatenate([pts, xyz_c], axis=-1)
    h = g3.reshape(Sb * K, Cp + 3)
    for li in range(layers_n):
        W = wrefs[2 * li][...]
        b = wrefs[2 * li + 1][...]
        h = jax.lax.dot_general(h, W, (((1,), (1,)), ((), ())),
                                preferred_element_type=jnp.float32)
        h = jax.nn.relu((h + b[None, :]) * BN_SCALE)
    Co = h.shape[-1]
    h = h.reshape(Sb, K, Co)
    o_ref[0] = jnp.max(h, axis=1)


def _sa_mlp(g3, ctr, cp, layers, s_block):
    B, S, K, C = g3.shape
    Co = layers[-1][0].shape[0]
    wargs = []
    in_specs = [
        pl.BlockSpec((1, s_block, K, C), lambda b, s: (b, s, 0, 0)),
        pl.BlockSpec((1, s_block, 3), lambda b, s: (b, s, 0)),
    ]
    for W, bb in layers:
        wargs += [W, bb]
        in_specs += [pl.BlockSpec(W.shape, lambda b, s: (0, 0)),
                     pl.BlockSpec(bb.shape, lambda b, s: (0,))]
    out = pl.pallas_call(
        functools.partial(_sa_mlp_body, K, cp, len(layers)),
        out_shape=jax.ShapeDtypeStruct((B, S, Co), jnp.float32),
        grid=(B, S // s_block),
        in_specs=in_specs,
        out_specs=pl.BlockSpec((1, s_block, Co), lambda b, s: (b, s, 0)),
    )(g3, ctr, *wargs)
    return out


def _sa3_fp3_body(*refs):
    rows_ref = refs[0]
    w = refs[1:11]
    l3_ref = refs[11]
    o_ref = refs[12]
    P = rows_ref.shape[1]
    h = rows_ref[0]
    for li in range(3):
        W = w[2 * li][...]
        b = w[2 * li + 1][...]
        h = jax.lax.dot_general(h, W, (((1,), (1,)), ((), ())),
                                preferred_element_type=jnp.float32)
        h = jax.nn.relu((h + b[None, :]) * BN_SCALE)
    l3 = jnp.max(h, axis=0)
    l3_ref[0, 0] = l3
    pts = rows_ref[0][:, 3:]
    h2 = jnp.concatenate([pts, jnp.broadcast_to(l3[None, :], (P, 1024))], axis=1)
    for li in range(2):
        W = w[6 + 2 * li][...]
        b = w[6 + 2 * li + 1][...]
        h2 = jax.lax.dot_general(h2, W, (((1,), (1,)), ((), ())),
                                 preferred_element_type=jnp.float32)
        h2 = jax.nn.relu((h2 + b[None, :]) * BN_SCALE)
    o_ref[0] = h2


def _sa3_fp3(l2_rows, sa3_layers, fp3_layers):
    B, P, C = l2_rows.shape
    wargs = []
    in_specs = [pl.BlockSpec((1, P, C), lambda b: (b, 0, 0))]
    for W, bb in sa3_layers + fp3_layers:
        wargs += [W, bb]
        in_specs += [pl.BlockSpec(W.shape, lambda b: (0, 0)),
                     pl.BlockSpec(bb.shape, lambda b: (0,))]
    l3, l2new = pl.pallas_call(
        _sa3_fp3_body,
        out_shape=[jax.ShapeDtypeStruct((B, 1, 1024), jnp.float32),
                   jax.ShapeDtypeStruct((B, P, 256), jnp.float32)],
        grid=(B,),
        in_specs=in_specs,
        out_specs=[pl.BlockSpec((1, 1, 1024), lambda b: (b, 0, 0)),
                   pl.BlockSpec((1, P, 256), lambda b: (b, 0, 0))],
    )(l2_rows, *wargs)
    return l3, l2new


def _fp_body(layers_n, *refs):
    xyz1_ref, xyz2t_ref, p1_ref, p2_ref = refs[:4]
    w = refs[4:4 + 2 * layers_n]
    o_ref = refs[4 + 2 * layers_n]
    x1 = xyz1_ref[0]
    x2t = xyz2t_ref[0]
    N = x1.shape[0]; S = x2t.shape[1]
    ns1 = jnp.sum(x1 * x1, axis=1, keepdims=True)
    ns2 = jnp.sum(x2t * x2t, axis=0, keepdims=True)
    mm = jax.lax.dot_general(x1, x2t, (((1,), (0,)), ((), ())),
                             preferred_element_type=jnp.float32)
    dists = (ns1 + ns2) - 2.0 * mm
    iota_s = jax.lax.broadcasted_iota(jnp.int32, (N, S), 1)
    cur = dists
    wmat = jnp.zeros((N, S), jnp.float32)
    wsum = jnp.zeros((N, 1), jnp.float32)
    onehots = []
    ws = []
    for _ in range(3):
        mk = jnp.min(cur, axis=1, keepdims=True)
        ik = jnp.min(jnp.where(cur == mk, iota_s, S), axis=1, keepdims=True)
        oh = iota_s == ik
        wk = 1.0 / (mk + 1e-8)
        onehots.append(oh)
        ws.append(wk)
        wsum = wsum + wk
        cur = jnp.where(oh, jnp.float32(np.inf), cur)
    for oh, wk in zip(onehots, ws):
        wmat = wmat + jnp.where(oh, (wk / wsum), 0.0)
    interp = jax.lax.dot_general(wmat, p2_ref[0], (((1,), (0,)), ((), ())),
                                 precision=jax.lax.Precision.HIGHEST,
                                 preferred_element_type=jnp.float32)
    h = jnp.concatenate([p1_ref[0], interp], axis=1)
    for li in range(layers_n):
        W = w[2 * li][...]
        b = w[2 * li + 1][...]
        h = jax.lax.dot_general(h, W, (((1,), (1,)), ((), ())),
                                preferred_element_type=jnp.float32)
        h = jax.nn.relu((h + b[None, :]) * BN_SCALE)
    o_ref[0] = h


def _fp(xyz1, xyz2, p1rows, p2rows, layers):
    B, N, _ = xyz1.shape
    S = xyz2.shape[1]
    C1 = p1rows.shape[2]; C2 = p2rows.shape[2]
    Co = layers[-1][0].shape[0]
    xyz2t = xyz2.transpose(0, 2, 1)
    wargs = []
    in_specs = [
        pl.BlockSpec((1, N, 3), lambda b: (b, 0, 0)),
        pl.BlockSpec((1, 3, S), lambda b: (b, 0, 0)),
        pl.BlockSpec((1, N, C1), lambda b: (b, 0, 0)),
        pl.BlockSpec((1, S, C2), lambda b: (b, 0, 0)),
    ]
    for W, bb in layers:
        wargs += [W, bb]
        in_specs += [pl.BlockSpec(W.shape, lambda b: (0, 0)),
                     pl.BlockSpec(bb.shape, lambda b: (0,))]
    out = pl.pallas_call(
        functools.partial(_fp_body, len(layers)),
        out_shape=jax.ShapeDtypeStruct((B, N, Co), jnp.float32),
        grid=(B,),
        in_specs=in_specs,
        out_specs=pl.BlockSpec((1, N, Co), lambda b: (b, 0, 0)),
    )(xyz1, xyz2t, p1rows, p2rows, *wargs)
    return out


def _head_body(x_ref, w1_ref, b1_ref, w2_ref, b2_ref, o_ref):
    x = x_ref[0]
    h = jax.lax.dot_general(x, w1_ref[...], (((1,), (1,)), ((), ())),
                            preferred_element_type=jnp.float32)
    h = jax.nn.relu((h + b1_ref[...][None, :]) * BN_SCALE)
    y = jax.lax.dot_general(h, w2_ref[...], (((1,), (1,)), ((), ())),
                            preferred_element_type=jnp.float32)
    y = y + b2_ref[...][None, :]
    m = jnp.max(y, axis=1, keepdims=True)
    sh = y - m
    lse = jnp.log(jnp.sum(jnp.exp(sh), axis=1, keepdims=True))
    o_ref[0] = sh - lse


def _head(rows, params):
    B, N, C = rows.shape
    W1, b1 = params['conv1']
    W2, b2 = params['conv2']
    out = pl.pallas_call(
        _head_body,
        out_shape=jax.ShapeDtypeStruct((B, N, 13), jnp.float32),
        grid=(B,),
        in_specs=[
            pl.BlockSpec((1, N, C), lambda b: (b, 0, 0)),
            pl.BlockSpec(W1.shape, lambda b: (0, 0)),
            pl.BlockSpec(b1.shape, lambda b: (0,)),
            pl.BlockSpec(W2.shape, lambda b: (0, 0)),
            pl.BlockSpec(b2.shape, lambda b: (0,)),
        ],
        out_specs=pl.BlockSpec((1, N, 13), lambda b: (b, 0, 0)),
    )(rows, W1, b1, W2, b2)
    return out


def _dq_body(x3_ref, nx_ref, o_ref):
    x3 = x3_ref[0]
    nx = nx_ref[0]
    nsx = jnp.sum(x3 * x3, axis=0, keepdims=True)
    nsn = jnp.sum(nx * nx, axis=1, keepdims=True)
    mm = jax.lax.dot_general(nx, x3, (((1,), (0,)), ((), ())),
                             preferred_element_type=jnp.float32)
    o_ref[0] = (nsn + nsx) - 2.0 * mm


def _dq(xyz3n, new_xyz):
    B, _, N = xyz3n.shape
    S = new_xyz.shape[1]
    return pl.pallas_call(
        _dq_body,
        out_shape=jax.ShapeDtypeStruct((B, S, N), jnp.float32),
        grid=(B,),
        in_specs=[pl.BlockSpec((1, 3, N), lambda b: (b, 0, 0)),
                  pl.BlockSpec((1, S, 3), lambda b: (b, 0, 0))],
        out_specs=pl.BlockSpec((1, S, N), lambda b: (b, 0, 0)),
    )(xyz3n, new_xyz)


_SC_NC, _SC_NS = 2, 16
_SC_NW = _SC_NC * _SC_NS


def _ballq_gather(dq, feats, radius, K):
    B, S, N = dq.shape
    C = feats.shape[-1]
    NCHUNK = N // 16
    Sw = S // _SC_NW
    r2 = jnp.float32(radius * radius)
    mesh = plsc.VectorSubcoreMesh(core_axis_name="c", subcore_axis_name="s")

    @functools.partial(
        pl.kernel,
        out_type=jax.ShapeDtypeStruct((B * S, K, C), jnp.float32),
        mesh=mesh,
        compiler_params=pltpu.CompilerParams(use_tc_tiling_on_sc=False,
                                             needs_layout_passes=False),
        scratch_types=[
            pltpu.VMEM((N,), jnp.float32),
            pltpu.VMEM((K + 16,), jnp.int32),
            pltpu.VMEM((K,), jnp.int32),
            pltpu.VMEM((K, C), jnp.float32),
            pltpu.SemaphoreType.DMA,
        ],
    )
    def k(dq_hbm, feats_hbm, out_hbm, drow, idxbuf, idxk, rows, sem):
        wid = lax.axis_index("s") * _SC_NC + lax.axis_index("c")
        iota16 = lax.broadcasted_iota(jnp.int32, (16,), 0)
        for b in range(B):
            def per_s(si, carry, b=b):
                s = wid * Sw + si
                g = b * S + s
                pltpu.sync_copy(dq_hbm.at[g], drow)

                def body(ch, ptr):
                    d16 = drow[pl.ds(ch * 16, 16)]
                    m = d16 <= r2
                    off = jnp.minimum(ptr, K)
                    plsc.store_compressed(idxbuf.at[pl.ds(off, 16)],
                                          iota16 + (ch * 16 + b * N), mask=m)
                    return ptr + jnp.sum(m.astype(jnp.int32))

                ptr = lax.fori_loop(0, NCHUNK, body, jnp.int32(0))
                cntf = jnp.minimum(ptr, K)
                c0 = idxbuf[pl.ds(0, 16)]
                first = jnp.min(
                    jnp.where(iota16 == 0, c0, jnp.int32(2 ** 30)), axis=0)
                for kk in range(K // 16):
                    cur = idxbuf[pl.ds(kk * 16, 16)]
                    slot = iota16 + kk * 16
                    idxk[pl.ds(kk * 16, 16)] = jnp.where(slot < cntf, cur,
                                                         first)
                pltpu.async_copy(feats_hbm.at[idxk], rows, sem).wait()
                pltpu.sync_copy(rows, out_hbm.at[g])
                return carry

            lax.fori_loop(0, Sw, per_s, 0)

    out = k(dq.reshape(B * S, N), feats.reshape(B * N, C))
    return out.reshape(B, S, K, C)


def _sa_msg(xyz_t, feat_rows, cp, npoint, radii, nsamples, branches,
            s_blocks):
    B, N, _ = xyz_t.shape
    new_xyz = _fps(xyz_t.transpose(0, 2, 1), npoint)
    dq = _dq(xyz_t.transpose(0, 2, 1), new_xyz)
    outs = []
    for radius, K, layers, sb in zip(radii, nsamples, branches, s_blocks):
        g3 = _ballq_gather(dq, feat_rows, radius, K)
        out = _sa_mlp(g3, new_xyz, cp, layers, sb)
        outs.append(out)
    return new_xyz, jnp.concatenate(outs, axis=-1)


def kernel(xyz, params):
    B, _, N = xyz.shape
    xyz_t = xyz.transpose(0, 2, 1)
    feat0 = jnp.concatenate(
        [xyz_t, xyz_t, jnp.zeros((B, N, 2), jnp.float32)], axis=-1)
    l1_xyz, l1_rows = _sa_msg(xyz_t, feat0, 3, 512, [0.1, 0.2, 0.4],
                              [32, 64, 128], params['sa1'],
                              [128, 128, 64])
    feat1 = jnp.concatenate(
        [l1_xyz, l1_rows, jnp.zeros((B, 512, 5), jnp.float32)], axis=-1)
    l2_xyz, l2_rows = _sa_msg(l1_xyz, feat1, 320, 128, [0.4, 0.8],
                              [64, 128], params['sa2'],
                              [64, 32])
    l2_all = jnp.concatenate([l2_xyz, l2_rows], axis=-1)
    l3_3d, l2new = _sa3_fp3(l2_all, params['sa3'], params['fp3'])
    l3 = l3_3d[:, 0]
    l1new = _fp(l1_xyz, l2_xyz, l1_rows, l2new, params['fp2'])
    l0new = _fp(xyz_t, l1_xyz, xyz_t, l1new, params['fp1'])
    logits = _head(l0new, params)
    return logits, l3[:, :, None]
